# baseline (speedup 1.0000x reference)
import functools

import jax
import jax.numpy as jnp
from jax import lax
from jax.experimental import pallas as pl
from jax.experimental.pallas import tpu as pltpu
from jax.experimental.pallas import tpu_sc as plsc

NC = 2
NS = 16
L = 16
NW = NC * NS

C = 8
NG = 3


def _emb_body(B, S, D, tok_hbm, table_hbm, pos_hbm, out_hbm,
              stage_v, sidx_v, rows_v, pos_v, g0, g1, g2, s0, s1, s2, psem):
    wid = lax.axis_index("s") * NC + lax.axis_index("c")
    SR = S // NW
    NJ = SR // C
    R = B * C
    sbase = wid * SR
    nvec = D // L
    gsem = (g0, g1, g2)
    ssem = (s0, s1, s2)

    def issue_pos(j):
        pltpu.async_copy(pos_hbm.at[pl.ds(sbase + j * C, C)], pos_v, psem)

    def wait_pos():
        pltpu.make_async_copy(pos_hbm.at[pl.ds(0, C)], pos_v, psem).wait()

    issue_pos(0)

    for k in range(B):
        pltpu.sync_copy(tok_hbm.at[pl.ds(k * S + sbase, SR)],
                        stage_v.at[pl.ds(k * SR, SR)])

    cshift = C.bit_length() - 1
    lane = lax.iota(jnp.int32, L)
    for h in range(R // L):
        l = lane + h * L
        k_l = lax.shift_right_logical(l, cshift)
        r_l = l & (C - 1)
        srow = k_l * S + r_l + sbase
        for j in range(NJ):
            sidx_v[j, pl.ds(h * L, L)] = srow + j * C

    def issue_gather(j, g):
        for k in range(B):
            pltpu.async_copy(
                table_hbm.at[stage_v.at[pl.ds(k * SR + j * C, C)]],
                rows_v.at[g, pl.ds(k * C, C)], gsem[g])

    def wait_gather(g):
        for k in range(B):
            pltpu.make_async_copy(
                table_hbm.at[stage_v.at[pl.ds(0, C)]],
                rows_v.at[g, pl.ds(k * C, C)], gsem[g]).wait()

    def issue_store(j, g):
        pltpu.async_copy(rows_v.at[g], out_hbm.at[sidx_v.at[j]], ssem[g])

    def wait_store(g):
        pltpu.make_async_copy(
            rows_v.at[g], out_hbm.at[sidx_v.at[0]], ssem[g]).wait()

    def body(j, g, store_wait=True, prefetch=True):
        gn = (g + 1) % NG
        wait_gather(g)
        if prefetch:
            if store_wait:
                wait_store(gn)
            issue_gather(j + 1, gn)
        wait_pos()

        @plsc.parallel_loop(0, C * nvec, unroll=8)
        def _add(i):
            r = i // nvec
            col = (i % nvec) * L
            v = pos_v[r, pl.ds(col, L)]
            for k in range(B):
                plsc.addupdate(rows_v.at[g, k * C + r, pl.ds(col, L)], v)

        if prefetch:
            issue_pos(j + 1)
        issue_store(j, g)

    issue_gather(0, 0)
    body(0, 0, store_wait=False)
    body(1, 1, store_wait=False)
    body(2, 2)
    body(3, 0)

    def super_body(t, carry):
        j = 4 + 3 * t
        body(j, 1)
        body(j + 1, 2)
        body(j + 2, 0)
        return carry

    lax.fori_loop(0, (NJ - 5) // NG, super_body, 0)

    body(NJ - 1, (NJ - 1) % NG, prefetch=False)
    for g in range(NG):
        wait_store(g)


def kernel(input_tokens, token_table, pos_table):
    B, S = input_tokens.shape
    _, D = token_table.shape
    N = B * S
    SR = S // NW
    NJ = SR // C

    mesh = plsc.VectorSubcoreMesh(core_axis_name="c", subcore_axis_name="s")
    k = functools.partial(
        pl.kernel,
        mesh=mesh,
        out_type=jax.ShapeDtypeStruct((N, D), jnp.float32),
        scratch_types=[
            pltpu.VMEM((B * SR,), jnp.int32),
            pltpu.VMEM((NJ, B * C), jnp.int32),
            pltpu.VMEM((NG, B * C, D), jnp.float32),
            pltpu.VMEM((C, D), jnp.float32),
            pltpu.SemaphoreType.DMA,
            pltpu.SemaphoreType.DMA,
            pltpu.SemaphoreType.DMA,
            pltpu.SemaphoreType.DMA,
            pltpu.SemaphoreType.DMA,
            pltpu.SemaphoreType.DMA,
            pltpu.SemaphoreType.DMA,
        ],
    )(functools.partial(_emb_body, B, S, D))

    tok_flat = input_tokens.reshape(-1).astype(jnp.int32)
    out = k(tok_flat, token_table, pos_table)
    return out.reshape(B, S, D)


# speedup vs baseline: 1.0066x; 1.0066x over previous
import functools

import jax
import jax.numpy as jnp
from jax import lax
from jax.experimental import pallas as pl
from jax.experimental.pallas import tpu as pltpu
from jax.experimental.pallas import tpu_sc as plsc

NC = 2
NS = 16
L = 16
NW = NC * NS

C = 8
NG = 3


def _emb_body(B, S, D, tok_hbm, table_hbm, pos_hbm, out_hbm,
              idx_v, rows_v, pos_v, g0, g1, g2, s0, s1, s2, psem):
    wid = lax.axis_index("s") * NC + lax.axis_index("c")
    SR = S // NW
    NJ = SR // C
    sbase = wid * SR
    nvec = D // L
    gsem = (g0, g1, g2)
    ssem = (s0, s1, s2)

    def issue_pos(j):
        pltpu.async_copy(pos_hbm.at[pl.ds(sbase + j * C, C)], pos_v, psem)

    def wait_pos():
        pltpu.make_async_copy(pos_hbm.at[pl.ds(0, C)], pos_v, psem).wait()

    def issue_gathers(j, g):
        for k in range(B):
            pltpu.async_copy(
                table_hbm.at[idx_v.at[pl.ds(k * SR + j * C, C)]],
                rows_v.at[g, pl.ds(k * C, C)], gsem[g])

    def wait_gathers(g):
        pltpu.make_async_copy(
            table_hbm.at[idx_v.at[pl.ds(0, B * C)]], rows_v.at[g],
            gsem[g]).wait()

    def issue_stores(j, g):
        for k in range(B):
            pltpu.async_copy(
                rows_v.at[g, pl.ds(k * C, C)],
                out_hbm.at[pl.ds(k * S + sbase + j * C, C)], ssem[g])

    def wait_stores(g):
        pltpu.make_async_copy(
            rows_v.at[g], out_hbm.at[pl.ds(0, B * C)], ssem[g]).wait()

    def body(j, g, store_wait=True, prefetch=True):
        gn = (g + 1) % NG
        if prefetch:
            if store_wait:
                wait_stores(gn)
            issue_gathers(j + 1, gn)
        wait_gathers(g)
        wait_pos()

        @plsc.parallel_loop(0, C * nvec, unroll=8)
        def _add(i):
            r = i // nvec
            col = (i % nvec) * L
            v = pos_v[r, pl.ds(col, L)]
            for k in range(B):
                plsc.addupdate(rows_v.at[g, k * C + r, pl.ds(col, L)], v)

        if prefetch:
            issue_pos(j + 1)
        issue_stores(j, g)

    issue_pos(0)
    for k in range(B):
        pltpu.sync_copy(tok_hbm.at[pl.ds(k * S + sbase, SR)],
                        idx_v.at[pl.ds(k * SR, SR)])

    issue_gathers(0, 0)
    body(0, 0, store_wait=False)
    body(1, 1, store_wait=False)
    body(2, 2)
    body(3, 0)

    def super_body(t, carry):
        j = 4 + 3 * t
        body(j, 1)
        body(j + 1, 2)
        body(j + 2, 0)
        return carry

    lax.fori_loop(0, (NJ - 5) // NG, super_body, 0)

    body(NJ - 1, (NJ - 1) % NG, prefetch=False)
    for g in range(NG):
        wait_stores(g)


def kernel(input_tokens, token_table, pos_table):
    B, S = input_tokens.shape
    _, D = token_table.shape
    N = B * S
    SR = S // NW

    mesh = plsc.VectorSubcoreMesh(core_axis_name="c", subcore_axis_name="s")
    k = functools.partial(
        pl.kernel,
        mesh=mesh,
        out_type=jax.ShapeDtypeStruct((N, D), jnp.float32),
        scratch_types=[
            pltpu.VMEM((B * SR,), jnp.int32),
            pltpu.VMEM((NG, B * C, D), jnp.float32),
            pltpu.VMEM((C, D), jnp.float32),
            pltpu.SemaphoreType.DMA,
            pltpu.SemaphoreType.DMA,
            pltpu.SemaphoreType.DMA,
            pltpu.SemaphoreType.DMA,
            pltpu.SemaphoreType.DMA,
            pltpu.SemaphoreType.DMA,
            pltpu.SemaphoreType.DMA,
        ],
    )(functools.partial(_emb_body, B, S, D))

    tok_flat = input_tokens.reshape(-1).astype(jnp.int32)
    out = k(tok_flat, token_table, pos_table)
    return out.reshape(B, S, D)


# speedup vs baseline: 1.0109x; 1.0043x over previous
import functools

import jax
import jax.numpy as jnp
from jax import lax
from jax.experimental import pallas as pl
from jax.experimental.pallas import tpu as pltpu
from jax.experimental.pallas import tpu_sc as plsc

NC = 2
NS = 16
L = 16
NW = NC * NS

C = 8
NG = 3


def _emb_body(B, S, D, tok_hbm, table_hbm, pos_hbm, out_hbm,
              idx_v, rows_v, pos_v, g0, g1, g2, s0, s1, s2, psem):
    wid = lax.axis_index("s") * NC + lax.axis_index("c")
    SR = S // NW
    NJ = SR // C
    sbase = wid * SR
    nvec = D // L
    gsem = (g0, g1, g2)
    ssem = (s0, s1, s2)

    def issue_pos(j):
        pltpu.async_copy(pos_hbm.at[pl.ds(sbase + j * C, C)], pos_v, psem)

    def wait_pos():
        pltpu.make_async_copy(pos_hbm.at[pl.ds(0, C)], pos_v, psem).wait()

    def issue_gathers(j, g):
        for k in range(B):
            pltpu.async_copy(
                table_hbm.at[idx_v.at[pl.ds(k * SR + j * C, C)]],
                rows_v.at[g, k], gsem[g])

    def wait_gathers(g):
        for k in range(B):
            pltpu.make_async_copy(
                table_hbm.at[idx_v.at[pl.ds(0, C)]], rows_v.at[g, k],
                gsem[g]).wait()

    def issue_stores(j, g):
        for k in range(B):
            pltpu.async_copy(
                rows_v.at[g, k],
                out_hbm.at[pl.ds(k * S + sbase + j * C, C)], ssem[g])

    def wait_stores(g):
        for k in range(B):
            pltpu.make_async_copy(
                rows_v.at[g, k], out_hbm.at[pl.ds(0, C)], ssem[g]).wait()

    def body(j, g, store_wait=True, prefetch=True):
        gn = (g + 1) % NG
        wait_gathers(g)
        if prefetch:
            if store_wait:
                wait_stores(gn)
            issue_gathers(j + 1, gn)
        wait_pos()

        @plsc.parallel_loop(0, C * nvec, unroll=8)
        def _add(i):
            r = i // nvec
            col = (i % nvec) * L
            v = pos_v[r, pl.ds(col, L)]
            for k in range(B):
                plsc.addupdate(rows_v.at[g, k, r, pl.ds(col, L)], v)

        if prefetch:
            issue_pos(j + 1)
        issue_stores(j, g)

    issue_pos(0)
    for k in range(B):
        pltpu.sync_copy(tok_hbm.at[pl.ds(k * S + sbase, SR)],
                        idx_v.at[pl.ds(k * SR, SR)])

    issue_gathers(0, 0)
    body(0, 0, store_wait=False)
    body(1, 1, store_wait=False)
    body(2, 2)
    body(3, 0)

    def super_body(t, carry):
        j = 4 + 3 * t
        body(j, 1)
        body(j + 1, 2)
        body(j + 2, 0)
        return carry

    lax.fori_loop(0, (NJ - 5) // NG, super_body, 0)

    body(NJ - 1, (NJ - 1) % NG, prefetch=False)
    for g in range(NG):
        wait_stores(g)


def kernel(input_tokens, token_table, pos_table):
    B, S = input_tokens.shape
    _, D = token_table.shape
    N = B * S
    SR = S // NW

    mesh = plsc.VectorSubcoreMesh(core_axis_name="c", subcore_axis_name="s")
    k = functools.partial(
        pl.kernel,
        mesh=mesh,
        out_type=jax.ShapeDtypeStruct((N, D), jnp.float32),
        scratch_types=[
            pltpu.VMEM((B * SR,), jnp.int32),
            pltpu.VMEM((NG, B, C, D), jnp.float32),
            pltpu.VMEM((C, D), jnp.float32),
            pltpu.SemaphoreType.DMA,
            pltpu.SemaphoreType.DMA,
            pltpu.SemaphoreType.DMA,
            pltpu.SemaphoreType.DMA,
            pltpu.SemaphoreType.DMA,
            pltpu.SemaphoreType.DMA,
            pltpu.SemaphoreType.DMA,
        ],
    )(functools.partial(_emb_body, B, S, D))

    tok_flat = input_tokens.reshape(-1).astype(jnp.int32)
    out = k(tok_flat, token_table, pos_table)
    return out.reshape(B, S, D)
